# Initial kernel scaffold; baseline (speedup 1.0000x reference)
import jax
import jax.numpy as jnp
from jax.experimental import pallas as pl


def kernel(x, edge_index, W1, b1, W2, b2):
    raise NotImplementedError("write your pallas kernel here")


# speedup vs baseline: 27.2095x; 27.2095x over previous
import functools

import jax
import jax.numpy as jnp
from jax import lax
from jax.experimental import pallas as pl
from jax.experimental.pallas import tpu as pltpu
from jax.experimental.pallas import tpu_sc as plsc

N_NODES = 10000
N_PAD = 10240
N_EDGES = 320000
D = 128
NC = 2
NS = 16
NW = NC * NS
EPW = N_EDGES // NW
CHUNK = 80
NJ = EPW // CHUNK
NJG = 25
NG = NJ // NJG
RPT = N_PAD // NS
ZR = 16

_mesh = plsc.VectorSubcoreMesh(core_axis_name="c", subcore_axis_name="s")


def _fill_vec(ref, n16, val):
    def body(i, _):
        ref[pl.ds(i * 16, 16)] = jnp.full((16,), val, jnp.float32)
        return 0
    lax.fori_loop(0, n16, body, 0)


@functools.partial(
    pl.kernel,
    out_type=jax.ShapeDtypeStruct((NC, N_PAD), jnp.float32),
    mesh=_mesh,
    scratch_types=[
        pltpu.VMEM_SHARED((N_PAD,), jnp.float32),
        pltpu.VMEM((NJ, CHUNK), jnp.int32),
        pltpu.VMEM((CHUNK,), jnp.float32),
        pltpu.VMEM((RPT,), jnp.float32),
    ],
)
def _sc_degree(dst_hbm, out_hbm, acc, dstv, onesv, zerov):
    c = lax.axis_index("c")
    s = lax.axis_index("s")
    wid = c * NS + s
    _fill_vec(onesv, CHUNK // 16, 1.0)
    _fill_vec(zerov, RPT // 16, 0.0)
    pltpu.sync_copy(zerov, acc.at[pl.ds(s * RPT, RPT)])
    pltpu.sync_copy(dst_hbm.at[wid], dstv)
    plsc.subcore_barrier()

    def step(j, _):
        pltpu.sync_copy(onesv, acc.at[dstv.at[j]], add=True)
        return 0
    lax.fori_loop(0, NJ, step, 0)
    plsc.subcore_barrier()
    pltpu.sync_copy(acc.at[pl.ds(s * RPT, RPT)],
                    out_hbm.at[c, pl.ds(s * RPT, RPT)])


@functools.partial(
    pl.kernel,
    out_type=jax.ShapeDtypeStruct((NC, N_PAD, D), jnp.float32),
    mesh=_mesh,
    scratch_types=[
        pltpu.VMEM_SHARED((N_PAD, D), jnp.float32),
        pltpu.VMEM((NJG, CHUNK), jnp.int32),
        pltpu.VMEM((NJG, CHUNK), jnp.int32),
        pltpu.VMEM((CHUNK, D), jnp.float32),
        pltpu.VMEM((CHUNK, D), jnp.float32),
        pltpu.VMEM((ZR, D), jnp.float32),
        pltpu.SemaphoreType.DMA,
        pltpu.SemaphoreType.DMA,
    ],
)
def _sc_scatter(xw_hbm, src_hbm, dst_hbm, out_hbm,
                acc, srcv, dstv, rows0, rows1, zerov, sem0, sem1):
    c = lax.axis_index("c")
    s = lax.axis_index("s")
    wid = c * NS + s

    def zbody(k, _):
        zerov[k // 8, pl.ds((k % 8) * 16, 16)] = jnp.zeros((16,), jnp.float32)
        return 0
    lax.fori_loop(0, ZR * 8, zbody, 0)

    def zcopy(t, _):
        pltpu.sync_copy(zerov, acc.at[pl.ds(s * RPT + t * ZR, ZR)])
        return 0
    lax.fori_loop(0, RPT // ZR, zcopy, 0)
    plsc.subcore_barrier()

    def group(g, _):
        pltpu.sync_copy(src_hbm.at[wid, g], srcv)
        pltpu.sync_copy(dst_hbm.at[wid, g], dstv)
        pltpu.async_copy(xw_hbm.at[srcv.at[0]], rows0, sem0)

        def step(jj, _):
            j2 = jj * 2
            pltpu.async_copy(xw_hbm.at[srcv.at[j2 + 1]], rows1, sem1)
            pltpu.make_async_copy(xw_hbm.at[srcv.at[j2]], rows0, sem0).wait()
            pltpu.sync_copy(rows0, acc.at[dstv.at[j2]], add=True)

            pltpu.async_copy(xw_hbm.at[srcv.at[j2 + 2]], rows0, sem0)
            pltpu.make_async_copy(xw_hbm.at[srcv.at[j2 + 1]], rows1,
                                  sem1).wait()
            pltpu.sync_copy(rows1, acc.at[dstv.at[j2 + 1]], add=True)
            return 0
        lax.fori_loop(0, (NJG - 1) // 2, step, 0)
        pltpu.make_async_copy(xw_hbm.at[srcv.at[NJG - 1]], rows0, sem0).wait()
        pltpu.sync_copy(rows0, acc.at[dstv.at[NJG - 1]], add=True)
        return 0
    lax.fori_loop(0, NG, group, 0)

    plsc.subcore_barrier()
    pltpu.sync_copy(acc.at[pl.ds(s * RPT, RPT)],
                    out_hbm.at[c, pl.ds(s * RPT, RPT)])


_GRID = N_PAD // 640


def _tc_prep1_body(x_ref, w_ref, d0_ref, d1_ref, xwp_ref, dis_ref):
    dis = lax.rsqrt(d0_ref[...] + d1_ref[...] + 1.0)
    xw = jnp.dot(x_ref[...], w_ref[...], preferred_element_type=jnp.float32)
    xwp_ref[...] = xw * dis
    dis_ref[...] = dis


def _tc_prep1(x_pad, W1, d0, d1):
    return pl.pallas_call(
        _tc_prep1_body,
        grid=(_GRID,),
        in_specs=[
            pl.BlockSpec((640, D), lambda i: (i, 0)),
            pl.BlockSpec((D, D), lambda i: (0, 0)),
            pl.BlockSpec((640, 1), lambda i: (i, 0)),
            pl.BlockSpec((640, 1), lambda i: (i, 0)),
        ],
        out_specs=[
            pl.BlockSpec((640, D), lambda i: (i, 0)),
            pl.BlockSpec((640, 1), lambda i: (i, 0)),
        ],
        out_shape=[
            jax.ShapeDtypeStruct((N_PAD, D), jnp.float32),
            jax.ShapeDtypeStruct((N_PAD, 1), jnp.float32),
        ],
    )(x_pad, W1, d0, d1)


def _tc_mid_body(p0_ref, p1_ref, xwp_ref, dis_ref, b_ref, w_ref, out_ref):
    dis = dis_ref[...]
    h = jax.nn.relu(dis * (p0_ref[...] + p1_ref[...] + xwp_ref[...])
                    + b_ref[...])
    out_ref[...] = jnp.dot(h, w_ref[...],
                           preferred_element_type=jnp.float32) * dis


def _tc_mid(p0, p1, xwp, dis, b1, W2):
    return pl.pallas_call(
        _tc_mid_body,
        grid=(_GRID,),
        in_specs=[
            pl.BlockSpec((640, D), lambda i: (i, 0)),
            pl.BlockSpec((640, D), lambda i: (i, 0)),
            pl.BlockSpec((640, D), lambda i: (i, 0)),
            pl.BlockSpec((640, 1), lambda i: (i, 0)),
            pl.BlockSpec((1, D), lambda i: (0, 0)),
            pl.BlockSpec((D, D), lambda i: (0, 0)),
        ],
        out_specs=pl.BlockSpec((640, D), lambda i: (i, 0)),
        out_shape=jax.ShapeDtypeStruct((N_PAD, D), jnp.float32),
    )(p0, p1, xwp, dis, b1, W2)


def _tc_final_body(p0_ref, p1_ref, xwp_ref, dis_ref, b_ref, out_ref):
    out_ref[...] = dis_ref[...] * (p0_ref[...] + p1_ref[...] + xwp_ref[...]) \
        + b_ref[...]


def _tc_final(p0, p1, xwp, dis, b2):
    return pl.pallas_call(
        _tc_final_body,
        grid=(_GRID,),
        in_specs=[
            pl.BlockSpec((640, D), lambda i: (i, 0)),
            pl.BlockSpec((640, D), lambda i: (i, 0)),
            pl.BlockSpec((640, D), lambda i: (i, 0)),
            pl.BlockSpec((640, 1), lambda i: (i, 0)),
            pl.BlockSpec((1, D), lambda i: (0, 0)),
        ],
        out_specs=pl.BlockSpec((640, D), lambda i: (i, 0)),
        out_shape=jax.ShapeDtypeStruct((N_PAD, D), jnp.float32),
    )(p0, p1, xwp, dis, b2)


def kernel(x, edge_index, W1, b1, W2, b2):
    e32 = edge_index.astype(jnp.int32)
    srcR = e32[0].reshape(NW, NG, NJG, CHUNK)
    dstR = e32[1].reshape(NW, NG, NJG, CHUNK)
    dstR3 = e32[1].reshape(NW, NJ, CHUNK)
    x_pad = jnp.pad(x, ((0, N_PAD - N_NODES), (0, 0)))
    b1r = b1.reshape(1, D)
    b2r = b2.reshape(1, D)

    degP = _sc_degree(dstR3)
    d0 = degP[0].reshape(N_PAD, 1)
    d1 = degP[1].reshape(N_PAD, 1)

    xw1p, dis = _tc_prep1(x_pad, W1, d0, d1)
    P1 = _sc_scatter(xw1p, srcR, dstR)
    xw2p = _tc_mid(P1[0], P1[1], xw1p, dis, b1r, W2)
    P2 = _sc_scatter(xw2p, srcR, dstR)
    out = _tc_final(P2[0], P2[1], xw2p, dis, b2r)
    return out[:N_NODES]


# speedup vs baseline: 27.3802x; 1.0063x over previous
import functools

import jax
import jax.numpy as jnp
from jax import lax
from jax.experimental import pallas as pl
from jax.experimental.pallas import tpu as pltpu
from jax.experimental.pallas import tpu_sc as plsc

N_NODES = 10000
N_PAD = 10240
N_EDGES = 320000
D = 128
NC = 2
NS = 16
NW = NC * NS
EPW = N_EDGES // NW
CHUNK = 80
NJ = EPW // CHUNK
NJG = 25
NG = NJ // NJG
RPT = N_PAD // NS
ZR = 16

_mesh = plsc.VectorSubcoreMesh(core_axis_name="c", subcore_axis_name="s")


def _fill_vec(ref, n16, val):
    def body(i, _):
        ref[pl.ds(i * 16, 16)] = jnp.full((16,), val, jnp.float32)
        return 0
    lax.fori_loop(0, n16, body, 0)


@functools.partial(
    pl.kernel,
    out_type=jax.ShapeDtypeStruct((NC, N_PAD), jnp.float32),
    mesh=_mesh,
    scratch_types=[
        pltpu.VMEM_SHARED((N_PAD,), jnp.float32),
        pltpu.VMEM((NJ, CHUNK), jnp.int32),
        pltpu.VMEM((CHUNK,), jnp.float32),
        pltpu.VMEM((RPT,), jnp.float32),
    ],
)
def _sc_degree(dst_hbm, out_hbm, acc, dstv, onesv, zerov):
    c = lax.axis_index("c")
    s = lax.axis_index("s")
    wid = c * NS + s
    _fill_vec(onesv, CHUNK // 16, 1.0)
    _fill_vec(zerov, RPT // 16, 0.0)
    pltpu.sync_copy(zerov, acc.at[pl.ds(s * RPT, RPT)])
    pltpu.sync_copy(dst_hbm.at[wid], dstv)
    plsc.subcore_barrier()

    def step(j, _):
        pltpu.sync_copy(onesv, acc.at[dstv.at[j]], add=True)
        return 0
    lax.fori_loop(0, NJ, step, 0)
    plsc.subcore_barrier()
    pltpu.sync_copy(acc.at[pl.ds(s * RPT, RPT)],
                    out_hbm.at[c, pl.ds(s * RPT, RPT)])


@functools.partial(
    pl.kernel,
    out_type=jax.ShapeDtypeStruct((NC, N_PAD, D), jnp.float32),
    mesh=_mesh,
    scratch_types=[
        pltpu.VMEM_SHARED((N_PAD, D), jnp.float32),
        pltpu.VMEM((NJG, CHUNK), jnp.int32),
        pltpu.VMEM((NJG, CHUNK), jnp.int32),
        pltpu.VMEM((CHUNK, D), jnp.float32),
        pltpu.VMEM((CHUNK, D), jnp.float32),
        pltpu.VMEM((ZR, D), jnp.float32),
        pltpu.SemaphoreType.DMA,
        pltpu.SemaphoreType.DMA,
    ],
)
def _sc_scatter(xw_hbm, src_hbm, dst_hbm, out_hbm,
                acc, srcv, dstv, rows0, rows1, zerov, sem0, sem1):
    c = lax.axis_index("c")
    s = lax.axis_index("s")
    wid = c * NS + s

    def zbody(k, _):
        zerov[k // 8, pl.ds((k % 8) * 16, 16)] = jnp.zeros((16,), jnp.float32)
        return 0
    lax.fori_loop(0, ZR * 8, zbody, 0)

    def zcopy(t, _):
        pltpu.sync_copy(zerov, acc.at[pl.ds(s * RPT + t * ZR, ZR)])
        return 0
    lax.fori_loop(0, RPT // ZR, zcopy, 0)
    plsc.subcore_barrier()

    def group(g, _):
        pltpu.sync_copy(src_hbm.at[wid, g], srcv)
        pltpu.sync_copy(dst_hbm.at[wid, g], dstv)
        pltpu.async_copy(xw_hbm.at[srcv.at[0]], rows0, sem0)

        def step(jj, _):
            j2 = jj * 2
            pltpu.async_copy(xw_hbm.at[srcv.at[j2 + 1]], rows1, sem1)
            pltpu.make_async_copy(xw_hbm.at[srcv.at[j2]], rows0, sem0).wait()
            pltpu.sync_copy(rows0, acc.at[dstv.at[j2]], add=True)

            pltpu.async_copy(xw_hbm.at[srcv.at[j2 + 2]], rows0, sem0)
            pltpu.make_async_copy(xw_hbm.at[srcv.at[j2 + 1]], rows1,
                                  sem1).wait()
            pltpu.sync_copy(rows1, acc.at[dstv.at[j2 + 1]], add=True)
            return 0
        lax.fori_loop(0, (NJG - 1) // 2, step, 0)
        pltpu.make_async_copy(xw_hbm.at[srcv.at[NJG - 1]], rows0, sem0).wait()
        pltpu.sync_copy(rows0, acc.at[dstv.at[NJG - 1]], add=True)
        return 0
    lax.fori_loop(0, NG, group, 0)

    plsc.subcore_barrier()
    pltpu.sync_copy(acc.at[pl.ds(s * RPT, RPT)],
                    out_hbm.at[c, pl.ds(s * RPT, RPT)])


_RB = 400
_GRID = N_NODES // _RB

_rb = pl.BlockSpec((_RB, D), lambda i: (i, 0))
_rb1 = pl.BlockSpec((_RB, 1), lambda i: (i, 0))
_p0 = pl.BlockSpec((1, _RB, D), lambda i: (0, i, 0))
_p1 = pl.BlockSpec((1, _RB, D), lambda i: (1, i, 0))
_d0 = pl.BlockSpec((1, _RB, 1), lambda i: (0, i, 0))
_d1 = pl.BlockSpec((1, _RB, 1), lambda i: (1, i, 0))
_wfull = pl.BlockSpec((D, D), lambda i: (0, 0))
_bfull = pl.BlockSpec((1, D), lambda i: (0, 0))


def _tc_mm1_body(x_ref, w_ref, out_ref):
    out_ref[...] = jnp.dot(x_ref[...], w_ref[...],
                           preferred_element_type=jnp.float32)


def _tc_mm1(x, W1):
    return pl.pallas_call(
        _tc_mm1_body,
        grid=(_GRID,),
        in_specs=[_rb, _wfull],
        out_specs=_rb,
        out_shape=jax.ShapeDtypeStruct((N_NODES, D), jnp.float32),
    )(x, W1)


def _tc_prep1_body(xw_ref, d0_ref, d1_ref, xwp_ref, dis_ref):
    dis = lax.rsqrt(d0_ref[0] + d1_ref[0] + 1.0)
    xwp_ref[...] = xw_ref[...] * dis
    dis_ref[...] = dis


def _tc_prep1(xw1, degP3):
    return pl.pallas_call(
        _tc_prep1_body,
        grid=(_GRID,),
        in_specs=[_rb, _d0, _d1],
        out_specs=[_rb, _rb1],
        out_shape=[
            jax.ShapeDtypeStruct((N_NODES, D), jnp.float32),
            jax.ShapeDtypeStruct((N_NODES, 1), jnp.float32),
        ],
    )(xw1, degP3, degP3)


def _tc_mid_body(p_ref, q_ref, xwp_ref, dis_ref, b_ref, w_ref, out_ref):
    dis = dis_ref[...]
    h = jax.nn.relu(dis * (p_ref[0] + q_ref[0] + xwp_ref[...]) + b_ref[...])
    out_ref[...] = jnp.dot(h, w_ref[...],
                           preferred_element_type=jnp.float32) * dis


def _tc_mid(P, xwp, dis, b1, W2):
    return pl.pallas_call(
        _tc_mid_body,
        grid=(_GRID,),
        in_specs=[_p0, _p1, _rb, _rb1, _bfull, _wfull],
        out_specs=_rb,
        out_shape=jax.ShapeDtypeStruct((N_NODES, D), jnp.float32),
    )(P, P, xwp, dis, b1, W2)


def _tc_final_body(p_ref, q_ref, xwp_ref, dis_ref, b_ref, out_ref):
    out_ref[...] = dis_ref[...] * (p_ref[0] + q_ref[0] + xwp_ref[...]) \
        + b_ref[...]


def _tc_final(P, xwp, dis, b2):
    return pl.pallas_call(
        _tc_final_body,
        grid=(_GRID,),
        in_specs=[_p0, _p1, _rb, _rb1, _bfull],
        out_specs=_rb,
        out_shape=jax.ShapeDtypeStruct((N_NODES, D), jnp.float32),
    )(P, P, xwp, dis, b2)


def kernel(x, edge_index, W1, b1, W2, b2):
    e32 = edge_index.astype(jnp.int32)
    srcR = e32[0].reshape(NW, NG, NJG, CHUNK)
    dstR = e32[1].reshape(NW, NG, NJG, CHUNK)
    dstR3 = e32[1].reshape(NW, NJ, CHUNK)
    b1r = b1.reshape(1, D)
    b2r = b2.reshape(1, D)

    degP3 = _sc_degree(dstR3).reshape(NC, N_PAD, 1)
    xw1 = _tc_mm1(x, W1)
    xw1p, dis = _tc_prep1(xw1, degP3)
    P1 = _sc_scatter(xw1p, srcR, dstR)
    xw2p = _tc_mid(P1, xw1p, dis, b1r, W2)
    P2 = _sc_scatter(xw2p, srcR, dstR)
    return _tc_final(P2, xw2p, dis, b2r)


# speedup vs baseline: 28.5117x; 1.0413x over previous
import functools

import jax
import jax.numpy as jnp
from jax import lax
from jax.experimental import pallas as pl
from jax.experimental.pallas import tpu as pltpu
from jax.experimental.pallas import tpu_sc as plsc

N_NODES = 10000
N_PAD = 10240
N_EDGES = 320000
D = 128
NC = 2
NS = 16
NW = NC * NS
EPW = N_EDGES // NW
CHUNK = 80
NJ = EPW // CHUNK
RPT = N_PAD // NS
ZR = 16

SCH = 64
EPW_P = 10240
E_PAD = NW * EPW_P
NSJ = EPW_P // SCH
SNG = 4
SGC = NSJ // SNG
NSLOT = 4

_mesh = plsc.VectorSubcoreMesh(core_axis_name="c", subcore_axis_name="s")


def _fill_vec(ref, n16, val):
    def body(i, _):
        ref[pl.ds(i * 16, 16)] = jnp.full((16,), val, jnp.float32)
        return 0
    lax.fori_loop(0, n16, body, 0)


@functools.partial(
    pl.kernel,
    out_type=jax.ShapeDtypeStruct((NC, N_PAD), jnp.float32),
    mesh=_mesh,
    scratch_types=[
        pltpu.VMEM_SHARED((N_PAD,), jnp.float32),
        pltpu.VMEM((NJ, CHUNK), jnp.int32),
        pltpu.VMEM((CHUNK,), jnp.float32),
        pltpu.VMEM((RPT,), jnp.float32),
    ],
)
def _sc_degree(dst_hbm, out_hbm, acc, dstv, onesv, zerov):
    c = lax.axis_index("c")
    s = lax.axis_index("s")
    wid = c * NS + s
    _fill_vec(onesv, CHUNK // 16, 1.0)
    _fill_vec(zerov, RPT // 16, 0.0)
    pltpu.sync_copy(zerov, acc.at[pl.ds(s * RPT, RPT)])
    pltpu.sync_copy(dst_hbm.at[wid], dstv)
    plsc.subcore_barrier()

    def step(j, _):
        pltpu.sync_copy(onesv, acc.at[dstv.at[j]], add=True)
        return 0
    lax.fori_loop(0, NJ, step, 0)
    plsc.subcore_barrier()
    pltpu.sync_copy(acc.at[pl.ds(s * RPT, RPT)],
                    out_hbm.at[c, pl.ds(s * RPT, RPT)])


@functools.partial(
    pl.kernel,
    out_type=jax.ShapeDtypeStruct((NC, N_PAD, D), jnp.float32),
    mesh=_mesh,
    scratch_types=[
        pltpu.VMEM_SHARED((N_PAD, D), jnp.float32),
        pltpu.VMEM((SGC, SCH), jnp.int32),
        pltpu.VMEM((SGC, SCH), jnp.int32),
        [pltpu.VMEM((SCH, D), jnp.float32) for _ in range(NSLOT)],
        pltpu.VMEM((ZR, D), jnp.float32),
        [pltpu.SemaphoreType.DMA for _ in range(NSLOT)],
        [pltpu.SemaphoreType.DMA for _ in range(NSLOT)],
    ],
)
def _sc_scatter(xw_hbm, src_hbm, dst_hbm, out_hbm,
                acc, srcv, dstv, rows, zerov, sg, ss):
    c = lax.axis_index("c")
    s = lax.axis_index("s")
    wid = c * NS + s

    def zbody(k, _):
        zerov[k // 8, pl.ds((k % 8) * 16, 16)] = jnp.zeros((16,), jnp.float32)
        return 0
    lax.fori_loop(0, ZR * 8, zbody, 0)

    def zcopy(t, _):
        pltpu.sync_copy(zerov, acc.at[pl.ds(s * RPT + t * ZR, ZR)])
        return 0
    lax.fori_loop(0, RPT // ZR, zcopy, 0)
    plsc.subcore_barrier()

    def start_g(u, k):
        pltpu.async_copy(xw_hbm.at[srcv.at[k]], rows[u], sg[u])

    def wait_g(u):
        pltpu.make_async_copy(xw_hbm.at[srcv.at[0]], rows[u], sg[u]).wait()

    def start_s(u, k):
        pltpu.async_copy(rows[u], acc.at[dstv.at[k]], ss[u], add=True)

    def wait_s(u):
        pltpu.make_async_copy(rows[u], acc.at[dstv.at[0]], ss[u]).wait()

    for g in range(SNG):
        pltpu.sync_copy(src_hbm.at[wid, g], srcv)
        pltpu.sync_copy(dst_hbm.at[wid, g], dstv)
        for u in range(NSLOT):
            if g > 0:
                wait_s(u)
            start_g(u, u)

        def quad(b, _):
            for u in range(NSLOT):
                wait_g(u)
                start_s(u, b * 4 + u)
            for u in range(NSLOT):
                wait_s(u)
                start_g(u, b * 4 + 4 + u)
            return 0
        lax.fori_loop(0, SGC // 4 - 1, quad, 0)
        for u in range(NSLOT):
            wait_g(u)
            start_s(u, SGC - 4 + u)
    for u in range(NSLOT):
        wait_s(u)

    plsc.subcore_barrier()
    pltpu.sync_copy(acc.at[pl.ds(s * RPT, RPT)],
                    out_hbm.at[c, pl.ds(s * RPT, RPT)])


_RB = 400
_GRID = N_NODES // _RB

_rb = pl.BlockSpec((_RB, D), lambda i: (i, 0))
_rb1 = pl.BlockSpec((_RB, 1), lambda i: (i, 0))
_p0 = pl.BlockSpec((1, _RB, D), lambda i: (0, i, 0))
_p1 = pl.BlockSpec((1, _RB, D), lambda i: (1, i, 0))
_d0 = pl.BlockSpec((1, _RB, 1), lambda i: (0, i, 0))
_d1 = pl.BlockSpec((1, _RB, 1), lambda i: (1, i, 0))
_wfull = pl.BlockSpec((D, D), lambda i: (0, 0))
_bfull = pl.BlockSpec((1, D), lambda i: (0, 0))


def _tc_mm1_body(x_ref, w_ref, out_ref):
    out_ref[...] = jnp.dot(x_ref[...], w_ref[...],
                           preferred_element_type=jnp.float32)


def _tc_mm1(x, W1):
    return pl.pallas_call(
        _tc_mm1_body,
        grid=(_GRID,),
        in_specs=[_rb, _wfull],
        out_specs=_rb,
        out_shape=jax.ShapeDtypeStruct((N_NODES, D), jnp.float32),
    )(x, W1)


def _tc_prep1_body(xw_ref, d0_ref, d1_ref, xwp_ref, dis_ref):
    dis = lax.rsqrt(d0_ref[0] + d1_ref[0] + 1.0)
    xwp_ref[...] = xw_ref[...] * dis
    dis_ref[...] = dis


def _tc_prep1(xw1, degP3):
    return pl.pallas_call(
        _tc_prep1_body,
        grid=(_GRID,),
        in_specs=[_rb, _d0, _d1],
        out_specs=[_rb, _rb1],
        out_shape=[
            jax.ShapeDtypeStruct((N_NODES, D), jnp.float32),
            jax.ShapeDtypeStruct((N_NODES, 1), jnp.float32),
        ],
    )(xw1, degP3, degP3)


def _tc_mid_body(p_ref, q_ref, xwp_ref, dis_ref, b_ref, w_ref, out_ref):
    dis = dis_ref[...]
    h = jax.nn.relu(dis * (p_ref[0] + q_ref[0] + xwp_ref[...]) + b_ref[...])
    out_ref[...] = jnp.dot(h, w_ref[...],
                           preferred_element_type=jnp.float32) * dis


def _tc_mid(P, xwp, dis, b1, W2):
    return pl.pallas_call(
        _tc_mid_body,
        grid=(_GRID,),
        in_specs=[_p0, _p1, _rb, _rb1, _bfull, _wfull],
        out_specs=_rb,
        out_shape=jax.ShapeDtypeStruct((N_NODES, D), jnp.float32),
    )(P, P, xwp, dis, b1, W2)


def _tc_final_body(p_ref, q_ref, xwp_ref, dis_ref, b_ref, out_ref):
    out_ref[...] = dis_ref[...] * (p_ref[0] + q_ref[0] + xwp_ref[...]) \
        + b_ref[...]


def _tc_final(P, xwp, dis, b2):
    return pl.pallas_call(
        _tc_final_body,
        grid=(_GRID,),
        in_specs=[_p0, _p1, _rb, _rb1, _bfull],
        out_specs=_rb,
        out_shape=jax.ShapeDtypeStruct((N_NODES, D), jnp.float32),
    )(P, P, xwp, dis, b2)


def kernel(x, edge_index, W1, b1, W2, b2):
    e32 = edge_index.astype(jnp.int32)
    pad_n = E_PAD - N_EDGES
    ar = jnp.arange(pad_n, dtype=jnp.int32)
    pad_src = (ar * 13) % N_NODES
    pad_dst = N_NODES + ar % (N_PAD - N_NODES)
    srcR = jnp.concatenate([e32[0], pad_src]).reshape(NW, SNG, SGC, SCH)
    dstR = jnp.concatenate([e32[1], pad_dst]).reshape(NW, SNG, SGC, SCH)
    dstR3 = e32[1].reshape(NW, NJ, CHUNK)
    b1r = b1.reshape(1, D)
    b2r = b2.reshape(1, D)

    degP3 = _sc_degree(dstR3).reshape(NC, N_PAD, 1)
    xw1 = _tc_mm1(x, W1)
    xw1p, dis = _tc_prep1(xw1, degP3)
    P1 = _sc_scatter(xw1p, srcR, dstR)
    xw2p = _tc_mid(P1, xw1p, dis, b1r, W2)
    P2 = _sc_scatter(xw2p, srcR, dstR)
    return _tc_final(P2, xw2p, dis, b2r)


# speedup vs baseline: 31.2611x; 1.0964x over previous
import functools

import jax
import jax.numpy as jnp
from jax import lax
from jax.experimental import pallas as pl
from jax.experimental.pallas import tpu as pltpu
from jax.experimental.pallas import tpu_sc as plsc

N_NODES = 10000
N_PAD = 10240
N_EDGES = 320000
D = 128
NC = 2
NS = 16
NW = NC * NS
EPW = N_EDGES // NW
CHUNK = 80
NJ = EPW // CHUNK
RPT = N_PAD // NS
ZR = 16

SCH = 64
EPW_P = 10240
E_PAD = NW * EPW_P
NSJ = EPW_P // SCH
SNG = 4
SGC = NSJ // SNG
NSLOT = 4

_mesh = plsc.VectorSubcoreMesh(core_axis_name="c", subcore_axis_name="s")


def _fill_vec(ref, n16, val):
    def body(i, _):
        ref[pl.ds(i * 16, 16)] = jnp.full((16,), val, jnp.float32)
        return 0
    lax.fori_loop(0, n16, body, 0)


@functools.partial(
    pl.kernel,
    out_type=jax.ShapeDtypeStruct((NC, N_PAD), jnp.float32),
    mesh=_mesh,
    scratch_types=[
        pltpu.VMEM_SHARED((N_PAD,), jnp.float32),
        pltpu.VMEM((SGC, SCH), jnp.int32),
        pltpu.VMEM((SCH,), jnp.float32),
        pltpu.VMEM((RPT,), jnp.float32),
    ],
)
def _sc_degree(dst_hbm, out_hbm, acc, dstv, onesv, zerov):
    c = lax.axis_index("c")
    s = lax.axis_index("s")
    wid = c * NS + s
    _fill_vec(onesv, SCH // 16, 1.0)
    _fill_vec(zerov, RPT // 16, 0.0)
    pltpu.sync_copy(zerov, acc.at[pl.ds(s * RPT, RPT)])
    plsc.subcore_barrier()

    def group(g, _):
        pltpu.sync_copy(dst_hbm.at[wid, g], dstv)

        def step(j, _):
            pltpu.sync_copy(onesv, acc.at[dstv.at[j]], add=True)
            return 0
        lax.fori_loop(0, SGC, step, 0)
        return 0
    lax.fori_loop(0, SNG, group, 0)
    plsc.subcore_barrier()
    pltpu.sync_copy(acc.at[pl.ds(s * RPT, RPT)],
                    out_hbm.at[c, pl.ds(s * RPT, RPT)])


@functools.partial(
    pl.kernel,
    out_type=jax.ShapeDtypeStruct((NC, N_PAD, D), jnp.float32),
    mesh=_mesh,
    scratch_types=[
        pltpu.VMEM_SHARED((N_PAD, D), jnp.float32),
        pltpu.VMEM((SGC, SCH), jnp.int32),
        pltpu.VMEM((SGC, SCH), jnp.int32),
        [pltpu.VMEM((SCH, D), jnp.float32) for _ in range(NSLOT)],
        pltpu.VMEM((ZR, D), jnp.float32),
        [pltpu.SemaphoreType.DMA for _ in range(NSLOT)],
        [pltpu.SemaphoreType.DMA for _ in range(NSLOT)],
    ],
)
def _sc_scatter(xw_hbm, src_hbm, dst_hbm, out_hbm,
                acc, srcv, dstv, rows, zerov, sg, ss):
    c = lax.axis_index("c")
    s = lax.axis_index("s")
    wid = c * NS + s

    def zbody(k, _):
        zerov[k // 8, pl.ds((k % 8) * 16, 16)] = jnp.zeros((16,), jnp.float32)
        return 0
    lax.fori_loop(0, ZR * 8, zbody, 0)

    def zcopy(t, _):
        pltpu.sync_copy(zerov, acc.at[pl.ds(s * RPT + t * ZR, ZR)])
        return 0
    lax.fori_loop(0, RPT // ZR, zcopy, 0)
    plsc.subcore_barrier()

    def start_g(u, k):
        pltpu.async_copy(xw_hbm.at[srcv.at[k]], rows[u], sg[u])

    def wait_g(u):
        pltpu.make_async_copy(xw_hbm.at[srcv.at[0]], rows[u], sg[u]).wait()

    def start_s(u, k):
        pltpu.async_copy(rows[u], acc.at[dstv.at[k]], ss[u], add=True)

    def wait_s(u):
        pltpu.make_async_copy(rows[u], acc.at[dstv.at[0]], ss[u]).wait()

    def quads(b, _):
        for u in range(NSLOT):
            wait_g(u)
            start_s(u, b * 4 + u)
        for u in range(NSLOT):
            wait_s(u)
            start_g(u, b * 4 + 4 + u)
        return 0

    def run_group(g, first):
        pltpu.sync_copy(src_hbm.at[wid, g], srcv)
        pltpu.sync_copy(dst_hbm.at[wid, g], dstv)
        for u in range(NSLOT):
            if not first:
                wait_s(u)
            start_g(u, u)
        lax.fori_loop(0, SGC // 4 - 1, quads, 0)
        for u in range(NSLOT):
            wait_g(u)
            start_s(u, SGC - 4 + u)

    run_group(0, True)

    def group(g, _):
        run_group(g, False)
        return 0
    lax.fori_loop(1, SNG, group, 0)
    for u in range(NSLOT):
        wait_s(u)

    plsc.subcore_barrier()
    pltpu.sync_copy(acc.at[pl.ds(s * RPT, RPT)],
                    out_hbm.at[c, pl.ds(s * RPT, RPT)])


_RB = 2000
_GRID = N_NODES // _RB

_rb = pl.BlockSpec((_RB, D), lambda i: (i, 0))
_rb1 = pl.BlockSpec((_RB, 1), lambda i: (i, 0))
_p0 = pl.BlockSpec((1, _RB, D), lambda i: (0, i, 0))
_p1 = pl.BlockSpec((1, _RB, D), lambda i: (1, i, 0))
_d0 = pl.BlockSpec((1, _RB, 1), lambda i: (0, i, 0))
_d1 = pl.BlockSpec((1, _RB, 1), lambda i: (1, i, 0))
_wfull = pl.BlockSpec((D, D), lambda i: (0, 0))
_bfull = pl.BlockSpec((1, D), lambda i: (0, 0))


def _tc_prep1_body(x_ref, w_ref, d0_ref, d1_ref, xwp_ref, dis_ref):
    dis = lax.rsqrt(d0_ref[0] + d1_ref[0] + 1.0)
    xwp_ref[...] = jnp.dot(x_ref[...], w_ref[...],
                           preferred_element_type=jnp.float32) * dis
    dis_ref[...] = dis


def _tc_prep1(x, W1, degP3):
    return pl.pallas_call(
        _tc_prep1_body,
        grid=(_GRID,),
        in_specs=[_rb, _wfull, _d0, _d1],
        out_specs=[_rb, _rb1],
        out_shape=[
            jax.ShapeDtypeStruct((N_NODES, D), jnp.float32),
            jax.ShapeDtypeStruct((N_NODES, 1), jnp.float32),
        ],
    )(x, W1, degP3, degP3)


def _tc_mid_body(p_ref, q_ref, xwp_ref, dis_ref, b_ref, w_ref, out_ref):
    dis = dis_ref[...]
    h = jax.nn.relu(dis * (p_ref[0] + q_ref[0] + xwp_ref[...]) + b_ref[...])
    out_ref[...] = jnp.dot(h, w_ref[...],
                           preferred_element_type=jnp.float32) * dis


def _tc_mid(P, xwp, dis, b1, W2):
    return pl.pallas_call(
        _tc_mid_body,
        grid=(_GRID,),
        in_specs=[_p0, _p1, _rb, _rb1, _bfull, _wfull],
        out_specs=_rb,
        out_shape=jax.ShapeDtypeStruct((N_NODES, D), jnp.float32),
    )(P, P, xwp, dis, b1, W2)


def _tc_final_body(p_ref, q_ref, xwp_ref, dis_ref, b_ref, out_ref):
    out_ref[...] = dis_ref[...] * (p_ref[0] + q_ref[0] + xwp_ref[...]) \
        + b_ref[...]


def _tc_final(P, xwp, dis, b2):
    return pl.pallas_call(
        _tc_final_body,
        grid=(_GRID,),
        in_specs=[_p0, _p1, _rb, _rb1, _bfull],
        out_specs=_rb,
        out_shape=jax.ShapeDtypeStruct((N_NODES, D), jnp.float32),
    )(P, P, xwp, dis, b2)


def kernel(x, edge_index, W1, b1, W2, b2):
    e32 = edge_index.astype(jnp.int32)
    pad_n = E_PAD - N_EDGES
    ar = jnp.arange(pad_n, dtype=jnp.int32)
    pad_src = (ar * 13) % N_NODES
    pad_dst = N_NODES + ar % (N_PAD - N_NODES)
    srcR = jnp.concatenate([e32[0], pad_src]).reshape(NW, SNG, SGC, SCH)
    dstR = jnp.concatenate([e32[1], pad_dst]).reshape(NW, SNG, SGC, SCH)
    b1r = b1.reshape(1, D)
    b2r = b2.reshape(1, D)

    degP3 = _sc_degree(dstR).reshape(NC, N_PAD, 1)
    xw1p, dis = _tc_prep1(x, W1, degP3)
    P1 = _sc_scatter(xw1p, srcR, dstR)
    xw2p = _tc_mid(P1, xw1p, dis, b1r, W2)
    P2 = _sc_scatter(xw2p, srcR, dstR)
    return _tc_final(P2, xw2p, dis, b2r)


# speedup vs baseline: 31.6870x; 1.0136x over previous
import functools

import jax
import jax.numpy as jnp
from jax import lax
from jax.experimental import pallas as pl
from jax.experimental.pallas import tpu as pltpu
from jax.experimental.pallas import tpu_sc as plsc

N_NODES = 10000
N_PAD = 10240
N_EDGES = 320000
D = 128
NC = 2
NS = 16
NW = NC * NS
EPW = N_EDGES // NW
CHUNK = 80
NJ = EPW // CHUNK
RPT = N_PAD // NS
ZR = 16

SCH = 64
EPW_P = 10240
E_PAD = NW * EPW_P
NSJ = EPW_P // SCH
SNG = 4
SGC = NSJ // SNG
NSLOT = 4

_mesh = plsc.VectorSubcoreMesh(core_axis_name="c", subcore_axis_name="s")


def _fill_vec(ref, n16, val):
    def body(i, _):
        ref[pl.ds(i * 16, 16)] = jnp.full((16,), val, jnp.float32)
        return 0
    lax.fori_loop(0, n16, body, 0)


@functools.partial(
    pl.kernel,
    out_type=jax.ShapeDtypeStruct((NC, N_PAD), jnp.float32),
    mesh=_mesh,
    scratch_types=[
        pltpu.VMEM_SHARED((N_PAD,), jnp.float32),
        pltpu.VMEM((SGC, SCH), jnp.int32),
        pltpu.VMEM((SCH,), jnp.float32),
        pltpu.VMEM((RPT,), jnp.float32),
    ],
)
def _sc_degree(dst_hbm, out_hbm, acc, dstv, onesv, zerov):
    c = lax.axis_index("c")
    s = lax.axis_index("s")
    wid = c * NS + s
    _fill_vec(onesv, SCH // 16, 1.0)
    _fill_vec(zerov, RPT // 16, 0.0)
    pltpu.sync_copy(zerov, acc.at[pl.ds(s * RPT, RPT)])
    plsc.subcore_barrier()

    def group(g, _):
        pltpu.sync_copy(dst_hbm.at[wid, g], dstv)

        def step(j, _):
            pltpu.sync_copy(onesv, acc.at[dstv.at[j]], add=True)
            return 0
        lax.fori_loop(0, SGC, step, 0)
        return 0
    lax.fori_loop(0, SNG, group, 0)
    plsc.subcore_barrier()
    pltpu.sync_copy(acc.at[pl.ds(s * RPT, RPT)],
                    out_hbm.at[c, pl.ds(s * RPT, RPT)])


@functools.partial(
    pl.kernel,
    out_type=jax.ShapeDtypeStruct((NC, N_PAD, D), jnp.float32),
    mesh=_mesh,
    scratch_types=[
        pltpu.VMEM_SHARED((N_PAD, D), jnp.float32),
        pltpu.VMEM((SGC, SCH), jnp.int32),
        pltpu.VMEM((SGC, SCH), jnp.int32),
        [pltpu.VMEM((SCH, D), jnp.float32) for _ in range(NSLOT)],
        pltpu.VMEM((ZR, D), jnp.float32),
        [pltpu.SemaphoreType.DMA for _ in range(NSLOT)],
        [pltpu.SemaphoreType.DMA for _ in range(NSLOT)],
    ],
)
def _sc_scatter(xw_hbm, src_hbm, dst_hbm, out_hbm,
                acc, srcv, dstv, rows, zerov, sg, ss):
    c = lax.axis_index("c")
    s = lax.axis_index("s")
    wid = c * NS + s

    def start_g(u, k):
        pltpu.async_copy(xw_hbm.at[srcv.at[k]], rows[u], sg[u])

    def wait_g(u):
        pltpu.make_async_copy(xw_hbm.at[srcv.at[0]], rows[u], sg[u]).wait()

    def start_s(u, k):
        pltpu.async_copy(rows[u], acc.at[dstv.at[k]], ss[u], add=True)

    def wait_s(u):
        pltpu.make_async_copy(rows[u], acc.at[dstv.at[0]], ss[u]).wait()

    def quads(b, _):
        for u in range(NSLOT):
            wait_g(u)
            start_s(u, b * 4 + u)
        for u in range(NSLOT):
            wait_s(u)
            start_g(u, b * 4 + 4 + u)
        return 0

    def run_group(g, first):
        pltpu.sync_copy(src_hbm.at[wid, g], srcv)
        pltpu.sync_copy(dst_hbm.at[wid, g], dstv)
        for u in range(NSLOT):
            if not first:
                wait_s(u)
            start_g(u, u)
        if first:
            def zbody(k, _):
                zerov[k // 8, pl.ds((k % 8) * 16, 16)] = \
                    jnp.zeros((16,), jnp.float32)
                return 0
            lax.fori_loop(0, ZR * 8, zbody, 0)

            def zcopy(t, _):
                pltpu.sync_copy(zerov, acc.at[pl.ds(s * RPT + t * ZR, ZR)])
                return 0
            lax.fori_loop(0, RPT // ZR, zcopy, 0)
            plsc.subcore_barrier()
        lax.fori_loop(0, SGC // 4 - 1, quads, 0)
        for u in range(NSLOT):
            wait_g(u)
            start_s(u, SGC - 4 + u)

    run_group(0, True)

    def group(g, _):
        run_group(g, False)
        return 0
    lax.fori_loop(1, SNG, group, 0)
    for u in range(NSLOT):
        wait_s(u)

    plsc.subcore_barrier()
    pltpu.sync_copy(acc.at[pl.ds(s * RPT, RPT)],
                    out_hbm.at[c, pl.ds(s * RPT, RPT)])


_RB = 2000
_GRID = N_NODES // _RB

_rb = pl.BlockSpec((_RB, D), lambda i: (i, 0))
_rb1 = pl.BlockSpec((_RB, 1), lambda i: (i, 0))
_p0 = pl.BlockSpec((1, _RB, D), lambda i: (0, i, 0))
_p1 = pl.BlockSpec((1, _RB, D), lambda i: (1, i, 0))
_d0 = pl.BlockSpec((1, _RB, 1), lambda i: (0, i, 0))
_d1 = pl.BlockSpec((1, _RB, 1), lambda i: (1, i, 0))
_wfull = pl.BlockSpec((D, D), lambda i: (0, 0))
_bfull = pl.BlockSpec((1, D), lambda i: (0, 0))


def _tc_prep1_body(x_ref, w_ref, d0_ref, d1_ref, xwp_ref, dis_ref):
    dis = lax.rsqrt(d0_ref[0] + d1_ref[0] + 1.0)
    xwp_ref[...] = jnp.dot(x_ref[...], w_ref[...],
                           preferred_element_type=jnp.float32) * dis
    dis_ref[...] = dis


def _tc_prep1(x, W1, degP):
    return pl.pallas_call(
        _tc_prep1_body,
        grid=(_GRID,),
        in_specs=[_rb, _wfull, _d0, _d1],
        out_specs=[_rb, _rb1],
        out_shape=[
            jax.ShapeDtypeStruct((N_NODES, D), jnp.float32),
            jax.ShapeDtypeStruct((N_NODES, 1), jnp.float32),
        ],
    )(x, W1, degP, degP)


def _tc_mid_body(p_ref, q_ref, xwp_ref, dis_ref, b_ref, w_ref, out_ref):
    dis = dis_ref[...]
    h = jax.nn.relu(dis * (p_ref[0] + q_ref[0] + xwp_ref[...]) + b_ref[...])
    out_ref[...] = jnp.dot(h, w_ref[...],
                           preferred_element_type=jnp.float32) * dis


def _tc_mid(P, xwp, dis, b1, W2):
    return pl.pallas_call(
        _tc_mid_body,
        grid=(_GRID,),
        in_specs=[_p0, _p1, _rb, _rb1, _bfull, _wfull],
        out_specs=_rb,
        out_shape=jax.ShapeDtypeStruct((N_NODES, D), jnp.float32),
    )(P, P, xwp, dis, b1, W2)


def _tc_final_body(p_ref, q_ref, xwp_ref, dis_ref, b_ref, out_ref):
    out_ref[...] = dis_ref[...] * (p_ref[0] + q_ref[0] + xwp_ref[...]) \
        + b_ref[...]


def _tc_final(P, xwp, dis, b2):
    return pl.pallas_call(
        _tc_final_body,
        grid=(_GRID,),
        in_specs=[_p0, _p1, _rb, _rb1, _bfull],
        out_specs=_rb,
        out_shape=jax.ShapeDtypeStruct((N_NODES, D), jnp.float32),
    )(P, P, xwp, dis, b2)


def kernel(x, edge_index, W1, b1, W2, b2):
    e32 = edge_index.astype(jnp.int32)
    pad_n = E_PAD - N_EDGES
    ar = jnp.arange(pad_n, dtype=jnp.int32)
    pad_src = (ar * 13) % N_NODES
    pad_dst = N_NODES + ar % (N_PAD - N_NODES)
    srcR = jnp.concatenate([e32[0], pad_src]).reshape(NW, SNG, SGC, SCH)
    dstR = jnp.concatenate([e32[1], pad_dst]).reshape(NW, SNG, SGC, SCH)
    b1r = b1.reshape(1, D)
    b2r = b2.reshape(1, D)

    degP = _sc_degree(dstR).reshape(NC, N_PAD, 1)
    xw1p, dis = _tc_prep1(x, W1, degP)
    P1 = _sc_scatter(xw1p, srcR, dstR)
    xw2p = _tc_mid(P1, xw1p, dis, b1r, W2)
    P2 = _sc_scatter(xw2p, srcR, dstR)
    return _tc_final(P2, xw2p, dis, b2r)


# speedup vs baseline: 32.2988x; 1.0193x over previous
import functools

import jax
import jax.numpy as jnp
from jax import lax
from jax.experimental import pallas as pl
from jax.experimental.pallas import tpu as pltpu
from jax.experimental.pallas import tpu_sc as plsc

N_NODES = 10000
N_PAD = 10240
N_EDGES = 320000
D = 128
NC = 2
NS = 16
NW = NC * NS
EPW = N_EDGES // NW
CHUNK = 80
NJ = EPW // CHUNK
RPT = N_PAD // NS
ZR = 16

SCH = 64
EPW_P = 10240
E_PAD = NW * EPW_P
NSJ = EPW_P // SCH
SNG = 4
SGC = NSJ // SNG
NSLOT = 4

DCH = 128
DNG = 4
DGC = EPW_P // (DCH * DNG)

_mesh = plsc.VectorSubcoreMesh(core_axis_name="c", subcore_axis_name="s")


def _fill_vec(ref, n16, val):
    def body(i, _):
        ref[pl.ds(i * 16, 16)] = jnp.full((16,), val, jnp.float32)
        return 0
    lax.fori_loop(0, n16, body, 0)


@functools.partial(
    pl.kernel,
    out_type=jax.ShapeDtypeStruct((NC, N_PAD), jnp.float32),
    mesh=_mesh,
    scratch_types=[
        pltpu.VMEM_SHARED((N_PAD,), jnp.float32),
        pltpu.VMEM((DGC, DCH), jnp.int32),
        pltpu.VMEM((DCH,), jnp.float32),
        pltpu.VMEM((RPT,), jnp.float32),
    ],
)
def _sc_degree(dst_hbm, out_hbm, acc, dstv, onesv, zerov):
    c = lax.axis_index("c")
    s = lax.axis_index("s")
    wid = c * NS + s
    _fill_vec(onesv, DCH // 16, 1.0)
    _fill_vec(zerov, RPT // 16, 0.0)
    pltpu.sync_copy(zerov, acc.at[pl.ds(s * RPT, RPT)])
    plsc.subcore_barrier()

    def group(g, _):
        pltpu.sync_copy(dst_hbm.at[wid, g], dstv)

        def step(j, _):
            pltpu.sync_copy(onesv, acc.at[dstv.at[j]], add=True)
            return 0
        lax.fori_loop(0, DGC, step, 0)
        return 0
    lax.fori_loop(0, DNG, group, 0)
    plsc.subcore_barrier()
    pltpu.sync_copy(acc.at[pl.ds(s * RPT, RPT)],
                    out_hbm.at[c, pl.ds(s * RPT, RPT)])


@functools.partial(
    pl.kernel,
    out_type=jax.ShapeDtypeStruct((NC, N_PAD, D), jnp.float32),
    mesh=_mesh,
    scratch_types=[
        pltpu.VMEM_SHARED((N_PAD, D), jnp.float32),
        pltpu.VMEM((SGC, SCH), jnp.int32),
        pltpu.VMEM((SGC, SCH), jnp.int32),
        [pltpu.VMEM((SCH, D), jnp.float32) for _ in range(NSLOT)],
        pltpu.VMEM((ZR, D), jnp.float32),
        [pltpu.SemaphoreType.DMA for _ in range(NSLOT)],
        [pltpu.SemaphoreType.DMA for _ in range(NSLOT)],
    ],
)
def _sc_scatter(xw_hbm, src_hbm, dst_hbm, out_hbm,
                acc, srcv, dstv, rows, zerov, sg, ss):
    c = lax.axis_index("c")
    s = lax.axis_index("s")
    wid = c * NS + s

    def start_g(u, k):
        pltpu.async_copy(xw_hbm.at[srcv.at[k]], rows[u], sg[u])

    def wait_g(u):
        pltpu.make_async_copy(xw_hbm.at[srcv.at[0]], rows[u], sg[u]).wait()

    def start_s(u, k):
        pltpu.async_copy(rows[u], acc.at[dstv.at[k]], ss[u], add=True)

    def wait_s(u):
        pltpu.make_async_copy(rows[u], acc.at[dstv.at[0]], ss[u]).wait()

    def quads(b, _):
        for u in range(NSLOT):
            wait_g(u)
            start_s(u, b * 4 + u)
        for u in range(NSLOT):
            wait_s(u)
            start_g(u, b * 4 + 4 + u)
        return 0

    def run_group(g, first):
        pltpu.sync_copy(src_hbm.at[wid, g], srcv)
        pltpu.sync_copy(dst_hbm.at[wid, g], dstv)
        for u in range(NSLOT):
            if not first:
                wait_s(u)
            start_g(u, u)
        if first:
            def zbody(k, _):
                zerov[k // 8, pl.ds((k % 8) * 16, 16)] = \
                    jnp.zeros((16,), jnp.float32)
                return 0
            lax.fori_loop(0, ZR * 8, zbody, 0)

            def zcopy(t, _):
                pltpu.sync_copy(zerov, acc.at[pl.ds(s * RPT + t * ZR, ZR)])
                return 0
            lax.fori_loop(0, RPT // ZR, zcopy, 0)
            plsc.subcore_barrier()
        lax.fori_loop(0, SGC // 4 - 1, quads, 0)
        for u in range(NSLOT):
            wait_g(u)
            start_s(u, SGC - 4 + u)

    run_group(0, True)

    def group(g, _):
        run_group(g, False)
        return 0
    lax.fori_loop(1, SNG, group, 0)
    for u in range(NSLOT):
        wait_s(u)

    plsc.subcore_barrier()
    pltpu.sync_copy(acc.at[pl.ds(s * RPT, RPT)],
                    out_hbm.at[c, pl.ds(s * RPT, RPT)])


_RB = 2000
_GRID = N_NODES // _RB

_rb = pl.BlockSpec((_RB, D), lambda i: (i, 0))
_rb1 = pl.BlockSpec((_RB, 1), lambda i: (i, 0))
_p0 = pl.BlockSpec((1, _RB, D), lambda i: (0, i, 0))
_p1 = pl.BlockSpec((1, _RB, D), lambda i: (1, i, 0))
_d0 = pl.BlockSpec((1, _RB, 1), lambda i: (0, i, 0))
_d1 = pl.BlockSpec((1, _RB, 1), lambda i: (1, i, 0))
_wfull = pl.BlockSpec((D, D), lambda i: (0, 0))
_bfull = pl.BlockSpec((1, D), lambda i: (0, 0))


def _tc_prep1_body(x_ref, w_ref, d0_ref, d1_ref, xwp_ref, dis_ref):
    dis = lax.rsqrt(d0_ref[0] + d1_ref[0] + 1.0)
    xwp_ref[...] = jnp.dot(x_ref[...], w_ref[...],
                           preferred_element_type=jnp.float32) * dis
    dis_ref[...] = dis


def _tc_prep1(x, W1, degP):
    return pl.pallas_call(
        _tc_prep1_body,
        grid=(_GRID,),
        in_specs=[_rb, _wfull, _d0, _d1],
        out_specs=[_rb, _rb1],
        out_shape=[
            jax.ShapeDtypeStruct((N_NODES, D), jnp.float32),
            jax.ShapeDtypeStruct((N_NODES, 1), jnp.float32),
        ],
    )(x, W1, degP, degP)


def _tc_mid_body(p_ref, q_ref, xwp_ref, dis_ref, b_ref, w_ref, out_ref):
    dis = dis_ref[...]
    h = jax.nn.relu(dis * (p_ref[0] + q_ref[0] + xwp_ref[...]) + b_ref[...])
    out_ref[...] = jnp.dot(h, w_ref[...],
                           preferred_element_type=jnp.float32) * dis


def _tc_mid(P, xwp, dis, b1, W2):
    return pl.pallas_call(
        _tc_mid_body,
        grid=(_GRID,),
        in_specs=[_p0, _p1, _rb, _rb1, _bfull, _wfull],
        out_specs=_rb,
        out_shape=jax.ShapeDtypeStruct((N_NODES, D), jnp.float32),
    )(P, P, xwp, dis, b1, W2)


def _tc_final_body(p_ref, q_ref, xwp_ref, dis_ref, b_ref, out_ref):
    out_ref[...] = dis_ref[...] * (p_ref[0] + q_ref[0] + xwp_ref[...]) \
        + b_ref[...]


def _tc_final(P, xwp, dis, b2):
    return pl.pallas_call(
        _tc_final_body,
        grid=(_GRID,),
        in_specs=[_p0, _p1, _rb, _rb1, _bfull],
        out_specs=_rb,
        out_shape=jax.ShapeDtypeStruct((N_NODES, D), jnp.float32),
    )(P, P, xwp, dis, b2)


def kernel(x, edge_index, W1, b1, W2, b2):
    e32 = edge_index.astype(jnp.int32)
    pad_n = E_PAD - N_EDGES
    ar = jnp.arange(pad_n, dtype=jnp.int32)
    pad_src = (ar * 13) % N_NODES
    pad_dst = N_NODES + ar % (N_PAD - N_NODES)
    srcR = jnp.concatenate([e32[0], pad_src]).reshape(NW, SNG, SGC, SCH)
    dstP = jnp.concatenate([e32[1], pad_dst])
    dstR = dstP.reshape(NW, SNG, SGC, SCH)
    dstF = dstP.reshape(NW, DNG, DGC, DCH)
    b1r = b1.reshape(1, D)
    b2r = b2.reshape(1, D)

    degP = _sc_degree(dstF).reshape(NC, N_PAD, 1)
    xw1p, dis = _tc_prep1(x, W1, degP)
    P1 = _sc_scatter(xw1p, srcR, dstR)
    xw2p = _tc_mid(P1, xw1p, dis, b1r, W2)
    P2 = _sc_scatter(xw2p, srcR, dstR)
    return _tc_final(P2, xw2p, dis, b2r)
